# baseline (speedup 1.0000x reference)
import functools

import jax
import jax.numpy as jnp
from jax import lax
from jax.experimental import pallas as pl
from jax.experimental.pallas import tpu as pltpu
from jax.experimental.pallas import tpu_sc as plsc

N_NODES = 10000
N_PAD = 10240
N_EDGES = 320000
H = 128

NC = 2
NS = 16
NW = NC * NS
CHUNK = 128
NCHUNKS = N_EDGES // CHUNK
BASE_CH = NCHUNKS // NW
EXTRA = NCHUNKS - BASE_CH * NW
RPT = N_PAD // NS


def _tables_body(h_ref, wr_ref, wc_ref, pr_ref, pc_ref):
    hv = h_ref[...]
    pr_ref[...] = jnp.dot(hv, wr_ref[...], preferred_element_type=jnp.float32)
    pc_ref[...] = jnp.dot(hv, wc_ref[...], preferred_element_type=jnp.float32)


def _make_tables(h, w_rt, w_ct):
    return pl.pallas_call(
        _tables_body,
        out_shape=(
            jax.ShapeDtypeStruct((N_NODES, H), jnp.float32),
            jax.ShapeDtypeStruct((N_NODES, H), jnp.float32),
        ),
    )(h, w_rt, w_ct)


def _gather_kernel(pr_hbm, pc_hbm, row_hbm, col_hbm, r_out, c_out,
                   idxr0, idxc0, idxr1, idxc1, rbuf0, cbuf0, rbuf1, cbuf1,
                   sg0, sg1, sw0, sw1):
    wid = lax.axis_index("s") * NC + lax.axis_index("c")

    idxr = (idxr0, idxr1)
    idxc = (idxc0, idxc1)
    rbuf = (rbuf0, rbuf1)
    cbuf = (cbuf0, cbuf1)
    sg = (sg0, sg1)
    sw = (sw0, sw1)

    def stage_idx(ci, p):
        ds = pl.ds((wid + ci * NW) * CHUNK, CHUNK)
        pltpu.sync_copy(row_hbm.at[ds], idxr[p])
        pltpu.sync_copy(col_hbm.at[ds], idxc[p])

    def start_gather(p):
        g1 = pltpu.async_copy(pr_hbm.at[idxr[p]], rbuf[p], sg[p])
        g2 = pltpu.async_copy(pc_hbm.at[idxc[p]], cbuf[p], sg[p])
        return g1, g2

    def start_wb(ci, p):
        ds = pl.ds((wid + ci * NW) * CHUNK, CHUNK)
        w1 = pltpu.async_copy(rbuf[p], r_out.at[ds], sw[p])
        w2 = pltpu.async_copy(cbuf[p], c_out.at[ds], sw[p])
        return w1, w2

    def pair(j, _):
        a = 2 * j
        b = 2 * j + 1
        stage_idx(a, 0)
        ga = start_gather(0)
        stage_idx(b, 1)
        ga[0].wait()
        ga[1].wait()
        wa = start_wb(a, 0)
        gb = start_gather(1)
        gb[0].wait()
        gb[1].wait()
        wb = start_wb(b, 1)
        wa[0].wait()
        wa[1].wait()
        wb[0].wait()
        wb[1].wait()
        return ()

    lax.fori_loop(0, BASE_CH // 2, pair, ())

    @pl.when(wid < EXTRA)
    def _tail():
        ci = BASE_CH
        stage_idx(ci, 0)
        g = start_gather(0)
        g[0].wait()
        g[1].wait()
        w = start_wb(ci, 0)
        w[0].wait()
        w[1].wait()


def _gather(pre_row, pre_col, row_idx, col_idx):
    k = functools.partial(
        pl.kernel,
        out_type=(
            jax.ShapeDtypeStruct((N_EDGES, H), jnp.float32),
            jax.ShapeDtypeStruct((N_EDGES, H), jnp.float32),
        ),
        mesh=plsc.VectorSubcoreMesh(core_axis_name="c", subcore_axis_name="s"),
        scratch_types=[
            pltpu.VMEM((CHUNK,), jnp.int32),
            pltpu.VMEM((CHUNK,), jnp.int32),
            pltpu.VMEM((CHUNK,), jnp.int32),
            pltpu.VMEM((CHUNK,), jnp.int32),
            pltpu.VMEM((CHUNK, H), jnp.float32),
            pltpu.VMEM((CHUNK, H), jnp.float32),
            pltpu.VMEM((CHUNK, H), jnp.float32),
            pltpu.VMEM((CHUNK, H), jnp.float32),
            pltpu.SemaphoreType.DMA,
            pltpu.SemaphoreType.DMA,
            pltpu.SemaphoreType.DMA,
            pltpu.SemaphoreType.DMA,
        ],
    )(_gather_kernel)
    return k(pre_row, pre_col, row_idx, col_idx)


def _mlp_body(r_ref, c_ref, ea_ref, cd3_ref, em_ref, w1ct_ref, b1_ref,
              w2t_ref, b2_ref, w3_ref, tx_ref, ty_ref, tz_ref):
    e1 = jnp.dot(ea_ref[...], w1ct_ref[...], preferred_element_type=jnp.float32)
    x1 = jax.nn.silu(r_ref[...] + c_ref[...] + e1 + b1_ref[...])
    x2 = jax.nn.silu(
        jnp.dot(x1, w2t_ref[...], preferred_element_type=jnp.float32)
        + b2_ref[...])
    m = jnp.sum(x2 * w3_ref[...], axis=1, keepdims=True)
    mscale = m * em_ref[...] * (1.0 / 100.0)
    trans = cd3_ref[...] * mscale
    tx_ref[...] = trans[:, 0:1]
    ty_ref[...] = trans[:, 1:2]
    tz_ref[...] = trans[:, 2:3]


def _mlp(r, c, edge_attr, cd3, edge_mask, w1ct, b1, w2t, b2, w3):
    EB = 2000
    grid = N_EDGES // EB
    return pl.pallas_call(
        _mlp_body,
        grid=(grid,),
        in_specs=[
            pl.BlockSpec((EB, H), lambda i: (i, 0)),
            pl.BlockSpec((EB, H), lambda i: (i, 0)),
            pl.BlockSpec((EB, 4), lambda i: (i, 0)),
            pl.BlockSpec((EB, 3), lambda i: (i, 0)),
            pl.BlockSpec((EB, 1), lambda i: (i, 0)),
            pl.BlockSpec((4, H), lambda i: (0, 0)),
            pl.BlockSpec((1, H), lambda i: (0, 0)),
            pl.BlockSpec((H, H), lambda i: (0, 0)),
            pl.BlockSpec((1, H), lambda i: (0, 0)),
            pl.BlockSpec((1, H), lambda i: (0, 0)),
        ],
        out_specs=[pl.BlockSpec((EB, 1), lambda i: (i, 0))] * 3,
        out_shape=tuple(jax.ShapeDtypeStruct((N_EDGES, 1), jnp.float32)
                        for _ in range(3)),
    )(r, c, edge_attr, cd3, edge_mask, w1ct, b1, w2t, b2, w3)


def _scatter_kernel(tx, ty, tz, row_hbm, zero_hbm, px, py, pz,
                    accx, accy, accz, bx, by, bz, ibuf):
    wid = lax.axis_index("s") * NC + lax.axis_index("c")
    nch = BASE_CH + jnp.where(wid < EXTRA, 1, 0)

    pltpu.sync_copy(zero_hbm, accx)
    pltpu.sync_copy(zero_hbm, accy)
    pltpu.sync_copy(zero_hbm, accz)
    iota = lax.iota(jnp.int32, 16)

    def chunk(i, _):
        off = (wid + i * NW) * CHUNK
        ds = pl.ds(off, CHUNK)
        pltpu.sync_copy(row_hbm.at[ds], ibuf)
        pltpu.sync_copy(tx.at[ds], bx)
        pltpu.sync_copy(ty.at[ds], by)
        pltpu.sync_copy(tz.at[ds], bz)

        def vreg(j, _):
            sl = pl.ds(j * 16, 16)
            ivec = ibuf[sl]
            xv = bx[sl]
            yv = by[sl]
            zv = bz[sl]
            for l in range(16):
                i = ivec[l]
                b = i & ~15
                m = iota == (i - b)
                w = pl.ds(b, 16)
                plsc.addupdate(accx.at[w], jnp.where(m, xv[l], 0.0))
                plsc.addupdate(accy.at[w], jnp.where(m, yv[l], 0.0))
                plsc.addupdate(accz.at[w], jnp.where(m, zv[l], 0.0))
            return ()

        lax.fori_loop(0, CHUNK // 16, vreg, ())
        return ()

    lax.fori_loop(0, nch, chunk, ())

    out_ds = pl.ds(wid * N_PAD, N_PAD)
    pltpu.sync_copy(accx, px.at[out_ds])
    pltpu.sync_copy(accy, py.at[out_ds])
    pltpu.sync_copy(accz, pz.at[out_ds])


def _scatter(tx, ty, tz, row_idx, zero_rows):
    k = functools.partial(
        pl.kernel,
        out_type=tuple(jax.ShapeDtypeStruct((NW * N_PAD,), jnp.float32)
                       for _ in range(3)),
        mesh=plsc.VectorSubcoreMesh(core_axis_name="c", subcore_axis_name="s"),
        scratch_types=[
            pltpu.VMEM((N_PAD,), jnp.float32),
            pltpu.VMEM((N_PAD,), jnp.float32),
            pltpu.VMEM((N_PAD,), jnp.float32),
            pltpu.VMEM((CHUNK,), jnp.float32),
            pltpu.VMEM((CHUNK,), jnp.float32),
            pltpu.VMEM((CHUNK,), jnp.float32),
            pltpu.VMEM((CHUNK,), jnp.int32),
        ],
    )(_scatter_kernel)
    return k(tx, ty, tz, row_idx, zero_rows)


def _final_body(coordt_ref, nmt_ref, px_ref, py_ref, pz_ref, out_ref):
    sx = jnp.sum(px_ref[...], axis=0)[:N_NODES]
    sy = jnp.sum(py_ref[...], axis=0)[:N_NODES]
    sz = jnp.sum(pz_ref[...], axis=0)[:N_NODES]
    agg = jnp.concatenate([sx[None, :], sy[None, :], sz[None, :]], axis=0)
    out_ref[...] = (coordt_ref[...] + agg) * nmt_ref[...]


def _finalize(coordt, nmt, px, py, pz):
    return pl.pallas_call(
        _final_body,
        out_shape=jax.ShapeDtypeStruct((3, N_NODES), jnp.float32),
    )(coordt, nmt, px, py, pz)


def kernel(h, coord, edge_index, coord_diff, edge_attr, node_mask, edge_mask,
           W1, b1, W2, b2, W3):
    row = edge_index[0].astype(jnp.int32)
    col = edge_index[1].astype(jnp.int32)
    w_rt = W1[:, :H].T
    w_ct = W1[:, H:2 * H].T
    w1ct = W1[:, 2 * H:].T
    zero_rows = jnp.zeros((N_PAD,), jnp.float32)

    pre_row, pre_col = _make_tables(h, w_rt, w_ct)
    r, c = _gather(pre_row, pre_col, row, col)
    tx, ty, tz = _mlp(r, c, edge_attr, coord_diff, edge_mask, w1ct,
                      b1.reshape(1, H), W2.T, b2.reshape(1, H), W3.reshape(1, H))
    px, py, pz = _scatter(tx.reshape(N_EDGES), ty.reshape(N_EDGES),
                          tz.reshape(N_EDGES), row, zero_rows)
    outt = _finalize(coord.T, node_mask.T,
                     px.reshape(NW, N_PAD), py.reshape(NW, N_PAD),
                     pz.reshape(NW, N_PAD))
    return outt.T


# speedup vs baseline: 1.1320x; 1.1320x over previous
import functools

import jax
import jax.numpy as jnp
from jax import lax
from jax.experimental import pallas as pl
from jax.experimental.pallas import tpu as pltpu
from jax.experimental.pallas import tpu_sc as plsc

N_NODES = 10000
N_PAD = 10240
N_EDGES = 320000
H = 128

NC = 2
NS = 16
NW = NC * NS
CHUNK = 128
NCHUNKS = N_EDGES // CHUNK
BASE_CH = NCHUNKS // NW
EXTRA = NCHUNKS - BASE_CH * NW
RPT = N_PAD // NS


def _tables_body(h_ref, wr_ref, wc_ref, pr_ref, pc_ref):
    hv = h_ref[...]
    pr_ref[...] = jnp.dot(hv, wr_ref[...], preferred_element_type=jnp.float32)
    pc_ref[...] = jnp.dot(hv, wc_ref[...], preferred_element_type=jnp.float32)


def _make_tables(h, w_rt, w_ct):
    return pl.pallas_call(
        _tables_body,
        out_shape=(
            jax.ShapeDtypeStruct((N_NODES, H), jnp.float32),
            jax.ShapeDtypeStruct((N_NODES, H), jnp.float32),
        ),
    )(h, w_rt, w_ct)


def _gather_kernel(pr_hbm, pc_hbm, row_hbm, col_hbm, r_out, c_out,
                   idxr0, idxc0, idxr1, idxc1, rbuf0, cbuf0, rbuf1, cbuf1,
                   sg0, sg1, sw0, sw1):
    wid = lax.axis_index("s") * NC + lax.axis_index("c")

    idxr = (idxr0, idxr1)
    idxc = (idxc0, idxc1)
    rbuf = (rbuf0, rbuf1)
    cbuf = (cbuf0, cbuf1)
    sg = (sg0, sg1)
    sw = (sw0, sw1)

    def stage_idx(ci, p):
        ds = pl.ds((wid + ci * NW) * CHUNK, CHUNK)
        pltpu.sync_copy(row_hbm.at[ds], idxr[p])
        pltpu.sync_copy(col_hbm.at[ds], idxc[p])

    def start_gather(p):
        g1 = pltpu.async_copy(pr_hbm.at[idxr[p]], rbuf[p], sg[p])
        g2 = pltpu.async_copy(pc_hbm.at[idxc[p]], cbuf[p], sg[p])
        return g1, g2

    def start_wb(ci, p):
        ds = pl.ds((wid + ci * NW) * CHUNK, CHUNK)
        w1 = pltpu.async_copy(rbuf[p], r_out.at[ds], sw[p])
        w2 = pltpu.async_copy(cbuf[p], c_out.at[ds], sw[p])
        return w1, w2

    def pair(j, _):
        a = 2 * j
        b = 2 * j + 1
        stage_idx(a, 0)
        ga = start_gather(0)
        stage_idx(b, 1)
        ga[0].wait()
        ga[1].wait()
        wa = start_wb(a, 0)
        gb = start_gather(1)
        gb[0].wait()
        gb[1].wait()
        wb = start_wb(b, 1)
        wa[0].wait()
        wa[1].wait()
        wb[0].wait()
        wb[1].wait()
        return ()

    lax.fori_loop(0, BASE_CH // 2, pair, ())

    @pl.when(wid < EXTRA)
    def _tail():
        ci = BASE_CH
        stage_idx(ci, 0)
        g = start_gather(0)
        g[0].wait()
        g[1].wait()
        w = start_wb(ci, 0)
        w[0].wait()
        w[1].wait()


def _gather(pre_row, pre_col, row_idx, col_idx):
    k = functools.partial(
        pl.kernel,
        out_type=(
            jax.ShapeDtypeStruct((N_EDGES, H), jnp.float32),
            jax.ShapeDtypeStruct((N_EDGES, H), jnp.float32),
        ),
        mesh=plsc.VectorSubcoreMesh(core_axis_name="c", subcore_axis_name="s"),
        scratch_types=[
            pltpu.VMEM((CHUNK,), jnp.int32),
            pltpu.VMEM((CHUNK,), jnp.int32),
            pltpu.VMEM((CHUNK,), jnp.int32),
            pltpu.VMEM((CHUNK,), jnp.int32),
            pltpu.VMEM((CHUNK, H), jnp.float32),
            pltpu.VMEM((CHUNK, H), jnp.float32),
            pltpu.VMEM((CHUNK, H), jnp.float32),
            pltpu.VMEM((CHUNK, H), jnp.float32),
            pltpu.SemaphoreType.DMA,
            pltpu.SemaphoreType.DMA,
            pltpu.SemaphoreType.DMA,
            pltpu.SemaphoreType.DMA,
        ],
    )(_gather_kernel)
    return k(pre_row, pre_col, row_idx, col_idx)


def _mlp_body(r_ref, c_ref, ea_ref, cd3_ref, em_ref, w1ct_ref, b1_ref,
              w2t_ref, b2_ref, w3m_ref, tx_ref, ty_ref, tz_ref):
    e1 = jnp.dot(ea_ref[...], w1ct_ref[...], preferred_element_type=jnp.float32)
    x1 = jax.nn.silu(r_ref[...] + c_ref[...] + e1 + b1_ref[...])
    x2 = jax.nn.silu(
        jnp.dot(x1.astype(jnp.bfloat16), w2t_ref[...],
                preferred_element_type=jnp.float32)
        + b2_ref[...])
    mf = jnp.dot(x2.astype(jnp.bfloat16), w3m_ref[...],
                 preferred_element_type=jnp.float32)
    m = mf[:, 0:1]
    mscale = m * em_ref[...] * (1.0 / 100.0)
    trans = cd3_ref[...] * mscale
    tx_ref[...] = trans[:, 0:1]
    ty_ref[...] = trans[:, 1:2]
    tz_ref[...] = trans[:, 2:3]


def _mlp(r, c, edge_attr, cd3, edge_mask, w1ct, b1, w2t, b2, w3m):
    EB = 4000
    grid = N_EDGES // EB
    return pl.pallas_call(
        _mlp_body,
        grid=(grid,),
        in_specs=[
            pl.BlockSpec((EB, H), lambda i: (i, 0)),
            pl.BlockSpec((EB, H), lambda i: (i, 0)),
            pl.BlockSpec((EB, 4), lambda i: (i, 0)),
            pl.BlockSpec((EB, 3), lambda i: (i, 0)),
            pl.BlockSpec((EB, 1), lambda i: (i, 0)),
            pl.BlockSpec((4, H), lambda i: (0, 0)),
            pl.BlockSpec((1, H), lambda i: (0, 0)),
            pl.BlockSpec((H, H), lambda i: (0, 0)),
            pl.BlockSpec((1, H), lambda i: (0, 0)),
            pl.BlockSpec((H, H), lambda i: (0, 0)),
        ],
        out_specs=[pl.BlockSpec((EB, 1), lambda i: (i, 0))] * 3,
        out_shape=tuple(jax.ShapeDtypeStruct((N_EDGES, 1), jnp.float32)
                        for _ in range(3)),
    )(r, c, edge_attr, cd3, edge_mask, w1ct, b1, w2t, b2, w3m)


def _scatter_kernel(tx, ty, tz, row_hbm, zero_hbm, px, py, pz,
                    accx, accy, accz, bx, by, bz, ibuf):
    wid = lax.axis_index("s") * NC + lax.axis_index("c")
    nch = BASE_CH + jnp.where(wid < EXTRA, 1, 0)

    pltpu.sync_copy(zero_hbm, accx)
    pltpu.sync_copy(zero_hbm, accy)
    pltpu.sync_copy(zero_hbm, accz)
    iota = lax.iota(jnp.int32, 16)

    def chunk(i, _):
        off = (wid + i * NW) * CHUNK
        ds = pl.ds(off, CHUNK)
        pltpu.sync_copy(row_hbm.at[ds], ibuf)
        pltpu.sync_copy(tx.at[ds], bx)
        pltpu.sync_copy(ty.at[ds], by)
        pltpu.sync_copy(tz.at[ds], bz)

        def vreg(j, _):
            sl = pl.ds(j * 16, 16)
            ivec = ibuf[sl]
            xv = bx[sl]
            yv = by[sl]
            zv = bz[sl]
            for l in range(16):
                i = ivec[l]
                b = i & ~15
                m = iota == (i - b)
                w = pl.ds(b, 16)
                plsc.addupdate(accx.at[w], jnp.where(m, xv[l], 0.0))
                plsc.addupdate(accy.at[w], jnp.where(m, yv[l], 0.0))
                plsc.addupdate(accz.at[w], jnp.where(m, zv[l], 0.0))
            return ()

        lax.fori_loop(0, CHUNK // 16, vreg, ())
        return ()

    lax.fori_loop(0, nch, chunk, ())

    out_ds = pl.ds(wid * N_PAD, N_PAD)
    pltpu.sync_copy(accx, px.at[out_ds])
    pltpu.sync_copy(accy, py.at[out_ds])
    pltpu.sync_copy(accz, pz.at[out_ds])


def _scatter(tx, ty, tz, row_idx, zero_rows):
    k = functools.partial(
        pl.kernel,
        out_type=tuple(jax.ShapeDtypeStruct((NW * N_PAD,), jnp.float32)
                       for _ in range(3)),
        mesh=plsc.VectorSubcoreMesh(core_axis_name="c", subcore_axis_name="s"),
        scratch_types=[
            pltpu.VMEM((N_PAD,), jnp.float32),
            pltpu.VMEM((N_PAD,), jnp.float32),
            pltpu.VMEM((N_PAD,), jnp.float32),
            pltpu.VMEM((CHUNK,), jnp.float32),
            pltpu.VMEM((CHUNK,), jnp.float32),
            pltpu.VMEM((CHUNK,), jnp.float32),
            pltpu.VMEM((CHUNK,), jnp.int32),
        ],
    )(_scatter_kernel)
    return k(tx, ty, tz, row_idx, zero_rows)


def _final_body(coordt_ref, nmt_ref, px_ref, py_ref, pz_ref, out_ref):
    sx = jnp.sum(px_ref[...], axis=0)[:N_NODES]
    sy = jnp.sum(py_ref[...], axis=0)[:N_NODES]
    sz = jnp.sum(pz_ref[...], axis=0)[:N_NODES]
    agg = jnp.concatenate([sx[None, :], sy[None, :], sz[None, :]], axis=0)
    out_ref[...] = (coordt_ref[...] + agg) * nmt_ref[...]


def _finalize(coordt, nmt, px, py, pz):
    return pl.pallas_call(
        _final_body,
        out_shape=jax.ShapeDtypeStruct((3, N_NODES), jnp.float32),
    )(coordt, nmt, px, py, pz)


def kernel(h, coord, edge_index, coord_diff, edge_attr, node_mask, edge_mask,
           W1, b1, W2, b2, W3):
    row = edge_index[0].astype(jnp.int32)
    col = edge_index[1].astype(jnp.int32)
    w_rt = W1[:, :H].T
    w_ct = W1[:, H:2 * H].T
    w1ct = W1[:, 2 * H:].T
    zero_rows = jnp.zeros((N_PAD,), jnp.float32)

    pre_row, pre_col = _make_tables(h, w_rt, w_ct)
    r, c = _gather(pre_row, pre_col, row, col)
    w2t_bf = W2.T.astype(jnp.bfloat16)
    w3m_bf = jnp.pad(W3.T, ((0, 0), (0, H - 1))).astype(jnp.bfloat16)
    tx, ty, tz = _mlp(r, c, edge_attr, coord_diff, edge_mask, w1ct,
                      b1.reshape(1, H), w2t_bf, b2.reshape(1, H), w3m_bf)
    px, py, pz = _scatter(tx.reshape(N_EDGES), ty.reshape(N_EDGES),
                          tz.reshape(N_EDGES), row, zero_rows)
    outt = _finalize(coord.T, node_mask.T,
                     px.reshape(NW, N_PAD), py.reshape(NW, N_PAD),
                     pz.reshape(NW, N_PAD))
    return outt.T


# speedup vs baseline: 1.2555x; 1.1091x over previous
import functools

import jax
import jax.numpy as jnp
from jax import lax
from jax.experimental import pallas as pl
from jax.experimental.pallas import tpu as pltpu
from jax.experimental.pallas import tpu_sc as plsc

N_NODES = 10000
N_PAD = 10240
N_EDGES = 320000
H = 128

NC = 2
NS = 16
NW = NC * NS
CHUNK = 128
NCHUNKS = N_EDGES // CHUNK
BASE_CH = NCHUNKS // NW
EXTRA = NCHUNKS - BASE_CH * NW
RPT = N_PAD // NS


def _tables_body(h_ref, wr_ref, wc_ref, pr_ref, pc_ref):
    hv = h_ref[...]
    pr_ref[...] = jnp.dot(hv, wr_ref[...], preferred_element_type=jnp.float32)
    pc_ref[...] = jnp.dot(hv, wc_ref[...], preferred_element_type=jnp.float32)


def _make_tables(h, w_rt, w_ct):
    return pl.pallas_call(
        _tables_body,
        out_shape=(
            jax.ShapeDtypeStruct((N_NODES, H), jnp.float32),
            jax.ShapeDtypeStruct((N_NODES, H), jnp.float32),
        ),
    )(h, w_rt, w_ct)


def _gather_kernel(pr_hbm, pc_hbm, row_hbm, col_hbm, u_out,
                   idxr0, idxc0, idxr1, idxc1, rbuf0, cbuf0, rbuf1, cbuf1,
                   sg0, sg1, sw0, sw1):
    wid = lax.axis_index("s") * NC + lax.axis_index("c")

    idxr = (idxr0, idxr1)
    idxc = (idxc0, idxc1)
    rbuf = (rbuf0, rbuf1)
    cbuf = (cbuf0, cbuf1)
    sg = (sg0, sg1)
    sw = (sw0, sw1)

    def stage_idx(ci, p):
        ds = pl.ds((wid + ci * NW) * CHUNK, CHUNK)
        pltpu.sync_copy(row_hbm.at[ds], idxr[p])
        pltpu.sync_copy(col_hbm.at[ds], idxc[p])

    def start_gather(p):
        g1 = pltpu.async_copy(pr_hbm.at[idxr[p]], rbuf[p], sg[p])
        g2 = pltpu.async_copy(pc_hbm.at[idxc[p]], cbuf[p], sg[p])
        return g1, g2

    def start_wb(ci, p):
        ds = pl.ds((wid + ci * NW) * CHUNK, CHUNK)
        w1 = pltpu.async_copy(rbuf[p], u_out.at[ds], sw[p])
        return (w1,)

    def add_pair(p):
        rb = rbuf[p]
        cb = cbuf[p]

        def rowadd(i, _):
            for j in range(H // 16):
                sl = pl.ds(j * 16, 16)
                rb[i, sl] = rb[i, sl] + cb[i, sl]
            return ()
        lax.fori_loop(0, CHUNK, rowadd, ())

    def pair(j, _):
        a = 2 * j
        b = 2 * j + 1
        stage_idx(a, 0)
        ga = start_gather(0)
        stage_idx(b, 1)
        ga[0].wait()
        ga[1].wait()
        gb = start_gather(1)
        add_pair(0)
        wa = start_wb(a, 0)
        gb[0].wait()
        gb[1].wait()
        add_pair(1)
        wb = start_wb(b, 1)
        wa[0].wait()
        wb[0].wait()
        return ()

    lax.fori_loop(0, BASE_CH // 2, pair, ())

    @pl.when(wid < EXTRA)
    def _tail():
        ci = BASE_CH
        stage_idx(ci, 0)
        g = start_gather(0)
        g[0].wait()
        g[1].wait()
        add_pair(0)
        w = start_wb(ci, 0)
        w[0].wait()


def _gather(pre_row, pre_col, row_idx, col_idx):
    k = functools.partial(
        pl.kernel,
        out_type=jax.ShapeDtypeStruct((N_EDGES, H), jnp.float32),
        mesh=plsc.VectorSubcoreMesh(core_axis_name="c", subcore_axis_name="s"),
        scratch_types=[
            pltpu.VMEM((CHUNK,), jnp.int32),
            pltpu.VMEM((CHUNK,), jnp.int32),
            pltpu.VMEM((CHUNK,), jnp.int32),
            pltpu.VMEM((CHUNK,), jnp.int32),
            pltpu.VMEM((CHUNK, H), jnp.float32),
            pltpu.VMEM((CHUNK, H), jnp.float32),
            pltpu.VMEM((CHUNK, H), jnp.float32),
            pltpu.VMEM((CHUNK, H), jnp.float32),
            pltpu.SemaphoreType.DMA,
            pltpu.SemaphoreType.DMA,
            pltpu.SemaphoreType.DMA,
            pltpu.SemaphoreType.DMA,
        ],
    )(_gather_kernel)
    return k(pre_row, pre_col, row_idx, col_idx)


def _mlp_body(u_ref, ea_ref, cd3_ref, em_ref, w1ct_ref, b1_ref,
              w2t_ref, b2_ref, w3m_ref, tx_ref, ty_ref, tz_ref):
    e1 = jnp.dot(ea_ref[...], w1ct_ref[...], preferred_element_type=jnp.float32)
    x1 = jax.nn.silu(u_ref[...] + e1 + b1_ref[...])
    x2 = jax.nn.silu(
        jnp.dot(x1.astype(jnp.bfloat16), w2t_ref[...],
                preferred_element_type=jnp.float32)
        + b2_ref[...])
    mf = jnp.dot(x2.astype(jnp.bfloat16), w3m_ref[...],
                 preferred_element_type=jnp.float32)
    m = mf[:, 0:1]
    mscale = m * em_ref[...] * (1.0 / 100.0)
    trans = cd3_ref[...] * mscale
    tx_ref[...] = trans[:, 0:1]
    ty_ref[...] = trans[:, 1:2]
    tz_ref[...] = trans[:, 2:3]


def _mlp(u, edge_attr, cd3, edge_mask, w1ct, b1, w2t, b2, w3m):
    EB = 4000
    grid = N_EDGES // EB
    return pl.pallas_call(
        _mlp_body,
        grid=(grid,),
        in_specs=[
            pl.BlockSpec((EB, H), lambda i: (i, 0)),
            pl.BlockSpec((EB, 4), lambda i: (i, 0)),
            pl.BlockSpec((EB, 3), lambda i: (i, 0)),
            pl.BlockSpec((EB, 1), lambda i: (i, 0)),
            pl.BlockSpec((4, H), lambda i: (0, 0)),
            pl.BlockSpec((1, H), lambda i: (0, 0)),
            pl.BlockSpec((H, H), lambda i: (0, 0)),
            pl.BlockSpec((1, H), lambda i: (0, 0)),
            pl.BlockSpec((H, H), lambda i: (0, 0)),
        ],
        out_specs=[pl.BlockSpec((EB, 1), lambda i: (i, 0))] * 3,
        out_shape=tuple(jax.ShapeDtypeStruct((N_EDGES, 1), jnp.float32)
                        for _ in range(3)),
    )(u, edge_attr, cd3, edge_mask, w1ct, b1, w2t, b2, w3m)


def _scatter_kernel(tx, ty, tz, row_hbm, zero_hbm, px, py, pz,
                    accx, accy, accz, bx, by, bz, ibuf):
    wid = lax.axis_index("s") * NC + lax.axis_index("c")
    nch = BASE_CH + jnp.where(wid < EXTRA, 1, 0)

    pltpu.sync_copy(zero_hbm, accx)
    pltpu.sync_copy(zero_hbm, accy)
    pltpu.sync_copy(zero_hbm, accz)
    iota = lax.iota(jnp.int32, 16)

    def chunk(i, _):
        off = (wid + i * NW) * CHUNK
        ds = pl.ds(off, CHUNK)
        pltpu.sync_copy(row_hbm.at[ds], ibuf)
        pltpu.sync_copy(tx.at[ds], bx)
        pltpu.sync_copy(ty.at[ds], by)
        pltpu.sync_copy(tz.at[ds], bz)

        def vreg(j, _):
            sl = pl.ds(j * 16, 16)
            ivec = ibuf[sl]
            xv = bx[sl]
            yv = by[sl]
            zv = bz[sl]
            for l in range(16):
                i = ivec[l]
                b = i & ~15
                m = iota == (i - b)
                w = pl.ds(b, 16)
                plsc.addupdate(accx.at[w], jnp.where(m, xv[l], 0.0))
                plsc.addupdate(accy.at[w], jnp.where(m, yv[l], 0.0))
                plsc.addupdate(accz.at[w], jnp.where(m, zv[l], 0.0))
            return ()

        lax.fori_loop(0, CHUNK // 16, vreg, ())
        return ()

    lax.fori_loop(0, nch, chunk, ())

    out_ds = pl.ds(wid * N_PAD, N_PAD)
    pltpu.sync_copy(accx, px.at[out_ds])
    pltpu.sync_copy(accy, py.at[out_ds])
    pltpu.sync_copy(accz, pz.at[out_ds])


def _scatter(tx, ty, tz, row_idx, zero_rows):
    k = functools.partial(
        pl.kernel,
        out_type=tuple(jax.ShapeDtypeStruct((NW * N_PAD,), jnp.float32)
                       for _ in range(3)),
        mesh=plsc.VectorSubcoreMesh(core_axis_name="c", subcore_axis_name="s"),
        scratch_types=[
            pltpu.VMEM((N_PAD,), jnp.float32),
            pltpu.VMEM((N_PAD,), jnp.float32),
            pltpu.VMEM((N_PAD,), jnp.float32),
            pltpu.VMEM((CHUNK,), jnp.float32),
            pltpu.VMEM((CHUNK,), jnp.float32),
            pltpu.VMEM((CHUNK,), jnp.float32),
            pltpu.VMEM((CHUNK,), jnp.int32),
        ],
    )(_scatter_kernel)
    return k(tx, ty, tz, row_idx, zero_rows)


def _final_body(coordt_ref, nmt_ref, px_ref, py_ref, pz_ref, out_ref):
    sx = jnp.sum(px_ref[...], axis=0)[:N_NODES]
    sy = jnp.sum(py_ref[...], axis=0)[:N_NODES]
    sz = jnp.sum(pz_ref[...], axis=0)[:N_NODES]
    agg = jnp.concatenate([sx[None, :], sy[None, :], sz[None, :]], axis=0)
    out_ref[...] = (coordt_ref[...] + agg) * nmt_ref[...]


def _finalize(coordt, nmt, px, py, pz):
    return pl.pallas_call(
        _final_body,
        out_shape=jax.ShapeDtypeStruct((3, N_NODES), jnp.float32),
    )(coordt, nmt, px, py, pz)


def kernel(h, coord, edge_index, coord_diff, edge_attr, node_mask, edge_mask,
           W1, b1, W2, b2, W3):
    row = edge_index[0].astype(jnp.int32)
    col = edge_index[1].astype(jnp.int32)
    w_rt = W1[:, :H].T
    w_ct = W1[:, H:2 * H].T
    w1ct = W1[:, 2 * H:].T
    zero_rows = jnp.zeros((N_PAD,), jnp.float32)

    pre_row, pre_col = _make_tables(h, w_rt, w_ct)
    u = _gather(pre_row, pre_col, row, col)
    w2t_bf = W2.T.astype(jnp.bfloat16)
    w3m_bf = jnp.pad(W3.T, ((0, 0), (0, H - 1))).astype(jnp.bfloat16)
    tx, ty, tz = _mlp(u, edge_attr, coord_diff, edge_mask, w1ct,
                      b1.reshape(1, H), w2t_bf, b2.reshape(1, H), w3m_bf)
    px, py, pz = _scatter(tx.reshape(N_EDGES), ty.reshape(N_EDGES),
                          tz.reshape(N_EDGES), row, zero_rows)
    outt = _finalize(coord.T, node_mask.T,
                     px.reshape(NW, N_PAD), py.reshape(NW, N_PAD),
                     pz.reshape(NW, N_PAD))
    return outt.T


# speedup vs baseline: 1.4159x; 1.1278x over previous
import functools

import jax
import jax.numpy as jnp
from jax import lax
from jax.experimental import pallas as pl
from jax.experimental.pallas import tpu as pltpu
from jax.experimental.pallas import tpu_sc as plsc

N_NODES = 10000
N_PAD = 10240
N_EDGES = 320000
H = 128

NC = 2
NS = 16
NW = NC * NS
CHUNK = 128
NCHUNKS = N_EDGES // CHUNK
BASE_CH = NCHUNKS // NW
EXTRA = NCHUNKS - BASE_CH * NW
RPT = N_PAD // NS


def _tables_body(h_ref, wr_ref, wc_ref, pr_ref, pc_ref):
    hv = h_ref[...]
    pr_ref[...] = jnp.dot(hv, wr_ref[...], preferred_element_type=jnp.float32)
    pc_ref[...] = jnp.dot(hv, wc_ref[...], preferred_element_type=jnp.float32)


def _make_tables(h, w_rt, w_ct):
    return pl.pallas_call(
        _tables_body,
        out_shape=(
            jax.ShapeDtypeStruct((N_NODES, H), jnp.float32),
            jax.ShapeDtypeStruct((N_NODES, H), jnp.float32),
        ),
    )(h, w_rt, w_ct)


def _gather_kernel(pr_hbm, pc_hbm, row_hbm, col_hbm, u_out,
                   idxr0, idxc0, idxr1, idxc1, rbuf0, cbuf0, rbuf1, cbuf1,
                   sg0, sg1, sw0, sw1):
    wid = lax.axis_index("s") * NC + lax.axis_index("c")

    idxr = (idxr0, idxr1)
    idxc = (idxc0, idxc1)
    rbuf = (rbuf0, rbuf1)
    cbuf = (cbuf0, cbuf1)
    sg = (sg0, sg1)
    sw = (sw0, sw1)

    def stage_idx(ci, p):
        ds = pl.ds((wid + ci * NW) * CHUNK, CHUNK)
        pltpu.sync_copy(row_hbm.at[ds], idxr[p])
        pltpu.sync_copy(col_hbm.at[ds], idxc[p])

    def start_gather(p):
        g1 = pltpu.async_copy(pr_hbm.at[idxr[p]], rbuf[p], sg[p])
        g2 = pltpu.async_copy(pc_hbm.at[idxc[p]], cbuf[p], sg[p])
        return g1, g2

    def start_wb(ci, p):
        ds = pl.ds((wid + ci * NW) * CHUNK, CHUNK)
        w1 = pltpu.async_copy(rbuf[p], u_out.at[ds], sw[p])
        return (w1,)

    def add_pair(p):
        rb = rbuf[p]
        cb = cbuf[p]

        def rowadd(i, _):
            for j in range(H // 16):
                sl = pl.ds(j * 16, 16)
                rb[i, sl] = rb[i, sl] + cb[i, sl]
            return ()
        lax.fori_loop(0, CHUNK, rowadd, ())

    def pair(j, _):
        a = 2 * j
        b = 2 * j + 1
        stage_idx(a, 0)
        ga = start_gather(0)
        stage_idx(b, 1)
        ga[0].wait()
        ga[1].wait()
        gb = start_gather(1)
        add_pair(0)
        wa = start_wb(a, 0)
        gb[0].wait()
        gb[1].wait()
        add_pair(1)
        wb = start_wb(b, 1)
        wa[0].wait()
        wb[0].wait()
        return ()

    lax.fori_loop(0, BASE_CH // 2, pair, ())

    @pl.when(wid < EXTRA)
    def _tail():
        ci = BASE_CH
        stage_idx(ci, 0)
        g = start_gather(0)
        g[0].wait()
        g[1].wait()
        add_pair(0)
        w = start_wb(ci, 0)
        w[0].wait()


def _gather(pre_row, pre_col, row_idx, col_idx):
    k = functools.partial(
        pl.kernel,
        out_type=jax.ShapeDtypeStruct((N_EDGES, H), jnp.float32),
        mesh=plsc.VectorSubcoreMesh(core_axis_name="c", subcore_axis_name="s"),
        scratch_types=[
            pltpu.VMEM((CHUNK,), jnp.int32),
            pltpu.VMEM((CHUNK,), jnp.int32),
            pltpu.VMEM((CHUNK,), jnp.int32),
            pltpu.VMEM((CHUNK,), jnp.int32),
            pltpu.VMEM((CHUNK, H), jnp.float32),
            pltpu.VMEM((CHUNK, H), jnp.float32),
            pltpu.VMEM((CHUNK, H), jnp.float32),
            pltpu.VMEM((CHUNK, H), jnp.float32),
            pltpu.SemaphoreType.DMA,
            pltpu.SemaphoreType.DMA,
            pltpu.SemaphoreType.DMA,
            pltpu.SemaphoreType.DMA,
        ],
    )(_gather_kernel)
    return k(pre_row, pre_col, row_idx, col_idx)


def _mlp_body(u_ref, ea_ref, cd3_ref, em_ref, w1ct_ref, b1_ref,
              w2t_ref, b2_ref, w3m_ref, tx_ref, ty_ref, tz_ref):
    e1 = jnp.dot(ea_ref[...], w1ct_ref[...], preferred_element_type=jnp.float32)
    x1 = jax.nn.silu(u_ref[...] + e1 + b1_ref[...])
    x2 = jax.nn.silu(
        jnp.dot(x1.astype(jnp.bfloat16), w2t_ref[...],
                preferred_element_type=jnp.float32)
        + b2_ref[...])
    mf = jnp.dot(x2.astype(jnp.bfloat16), w3m_ref[...],
                 preferred_element_type=jnp.float32)
    m = mf[:, 0:1]
    mscale = m * em_ref[...] * (1.0 / 100.0)
    trans = cd3_ref[...] * mscale
    tx_ref[...] = trans[:, 0:1]
    ty_ref[...] = trans[:, 1:2]
    tz_ref[...] = trans[:, 2:3]


def _mlp(u, edge_attr, cd3, edge_mask, w1ct, b1, w2t, b2, w3m):
    EB = 4000
    grid = N_EDGES // EB
    return pl.pallas_call(
        _mlp_body,
        grid=(grid,),
        in_specs=[
            pl.BlockSpec((EB, H), lambda i: (i, 0)),
            pl.BlockSpec((EB, 4), lambda i: (i, 0)),
            pl.BlockSpec((EB, 3), lambda i: (i, 0)),
            pl.BlockSpec((EB, 1), lambda i: (i, 0)),
            pl.BlockSpec((4, H), lambda i: (0, 0)),
            pl.BlockSpec((1, H), lambda i: (0, 0)),
            pl.BlockSpec((H, H), lambda i: (0, 0)),
            pl.BlockSpec((1, H), lambda i: (0, 0)),
            pl.BlockSpec((H, H), lambda i: (0, 0)),
        ],
        out_specs=[pl.BlockSpec((EB, 1), lambda i: (i, 0))] * 3,
        out_shape=tuple(jax.ShapeDtypeStruct((N_EDGES, 1), jnp.float32)
                        for _ in range(3)),
    )(u, edge_attr, cd3, edge_mask, w1ct, b1, w2t, b2, w3m)


def _scatter_kernel(tx, ty, tz, row_hbm, zero_hbm, px, py, pz,
                    accx, accy, accz, bx0, by0, bz0, ib0,
                    bx1, by1, bz1, ib1, sp0, sp1):
    wid = lax.axis_index("s") * NC + lax.axis_index("c")

    pltpu.sync_copy(zero_hbm, accx)
    pltpu.sync_copy(zero_hbm, accy)
    pltpu.sync_copy(zero_hbm, accz)
    iota = lax.iota(jnp.int32, 16)

    bx = (bx0, bx1)
    by = (by0, by1)
    bz = (bz0, bz1)
    ib = (ib0, ib1)
    sp = (sp0, sp1)

    def stage(ci, p):
        ds = pl.ds((wid + ci * NW) * CHUNK, CHUNK)
        return (pltpu.async_copy(row_hbm.at[ds], ib[p], sp[p]),
                pltpu.async_copy(tx.at[ds], bx[p], sp[p]),
                pltpu.async_copy(ty.at[ds], by[p], sp[p]),
                pltpu.async_copy(tz.at[ds], bz[p], sp[p]))

    def do_scatter(p):
        def vreg(j, _):
            sl = pl.ds(j * 16, 16)
            ivec = ib[p][sl]
            xv = bx[p][sl]
            yv = by[p][sl]
            zv = bz[p][sl]
            for l in range(16):
                i = ivec[l]
                b = i & ~15
                m = iota == (i - b)
                w = pl.ds(b, 16)
                plsc.addupdate(accx.at[w], jnp.where(m, xv[l], 0.0))
                plsc.addupdate(accy.at[w], jnp.where(m, yv[l], 0.0))
                plsc.addupdate(accz.at[w], jnp.where(m, zv[l], 0.0))
            return ()
        lax.fori_loop(0, CHUNK // 16, vreg, ())

    def pair(j, _):
        a = 2 * j
        b = 2 * j + 1
        sa = stage(a, 0)
        sb = stage(b, 1)
        for cp in sa:
            cp.wait()
        do_scatter(0)
        for cp in sb:
            cp.wait()
        do_scatter(1)
        return ()

    lax.fori_loop(0, BASE_CH // 2, pair, ())

    @pl.when(wid < EXTRA)
    def _tail():
        st = stage(BASE_CH, 0)
        for cp in st:
            cp.wait()
        do_scatter(0)

    out_ds = pl.ds(wid * N_PAD, N_PAD)
    pltpu.sync_copy(accx, px.at[out_ds])
    pltpu.sync_copy(accy, py.at[out_ds])
    pltpu.sync_copy(accz, pz.at[out_ds])


def _scatter(tx, ty, tz, row_idx, zero_rows):
    k = functools.partial(
        pl.kernel,
        out_type=tuple(jax.ShapeDtypeStruct((NW * N_PAD,), jnp.float32)
                       for _ in range(3)),
        mesh=plsc.VectorSubcoreMesh(core_axis_name="c", subcore_axis_name="s"),
        scratch_types=[
            pltpu.VMEM((N_PAD,), jnp.float32),
            pltpu.VMEM((N_PAD,), jnp.float32),
            pltpu.VMEM((N_PAD,), jnp.float32),
            pltpu.VMEM((CHUNK,), jnp.float32),
            pltpu.VMEM((CHUNK,), jnp.float32),
            pltpu.VMEM((CHUNK,), jnp.float32),
            pltpu.VMEM((CHUNK,), jnp.int32),
            pltpu.VMEM((CHUNK,), jnp.float32),
            pltpu.VMEM((CHUNK,), jnp.float32),
            pltpu.VMEM((CHUNK,), jnp.float32),
            pltpu.VMEM((CHUNK,), jnp.int32),
            pltpu.SemaphoreType.DMA,
            pltpu.SemaphoreType.DMA,
        ],
    )(_scatter_kernel)
    return k(tx, ty, tz, row_idx, zero_rows)


def _final_body(coordt_ref, nmt_ref, px_ref, py_ref, pz_ref, out_ref):
    sx = jnp.sum(px_ref[...], axis=0)[:N_NODES]
    sy = jnp.sum(py_ref[...], axis=0)[:N_NODES]
    sz = jnp.sum(pz_ref[...], axis=0)[:N_NODES]
    agg = jnp.concatenate([sx[None, :], sy[None, :], sz[None, :]], axis=0)
    out_ref[...] = (coordt_ref[...] + agg) * nmt_ref[...]


def _finalize(coordt, nmt, px, py, pz):
    return pl.pallas_call(
        _final_body,
        out_shape=jax.ShapeDtypeStruct((3, N_NODES), jnp.float32),
    )(coordt, nmt, px, py, pz)


def kernel(h, coord, edge_index, coord_diff, edge_attr, node_mask, edge_mask,
           W1, b1, W2, b2, W3):
    row = edge_index[0].astype(jnp.int32)
    col = edge_index[1].astype(jnp.int32)
    w_rt = W1[:, :H].T
    w_ct = W1[:, H:2 * H].T
    w1ct = W1[:, 2 * H:].T
    zero_rows = jnp.zeros((N_PAD,), jnp.float32)

    pre_row, pre_col = _make_tables(h, w_rt, w_ct)
    u = _gather(pre_row, pre_col, row, col)
    w2t_bf = W2.T.astype(jnp.bfloat16)
    w3m_bf = jnp.pad(W3.T, ((0, 0), (0, H - 1))).astype(jnp.bfloat16)
    tx, ty, tz = _mlp(u, edge_attr, coord_diff, edge_mask, w1ct,
                      b1.reshape(1, H), w2t_bf, b2.reshape(1, H), w3m_bf)
    px, py, pz = _scatter(tx.reshape(N_EDGES), ty.reshape(N_EDGES),
                          tz.reshape(N_EDGES), row, zero_rows)
    outt = _finalize(coord.T, node_mask.T,
                     px.reshape(NW, N_PAD), py.reshape(NW, N_PAD),
                     pz.reshape(NW, N_PAD))
    return outt.T
